# Initial kernel scaffold; baseline (speedup 1.0000x reference)
import jax
import jax.numpy as jnp
from jax.experimental import pallas as pl


def kernel(features_1, positions_1, batch_1, features_2, positions_2, batch_2, W1, b1, g1, be1, W2, b2, g2, be2):
    raise NotImplementedError("write your pallas kernel here")


# speedup vs baseline: 7.8507x; 7.8507x over previous
import functools

import jax
import jax.numpy as jnp
from jax.experimental import pallas as pl

N = 4096
L = 16384
IN1 = 512
IN2 = 256
OUT = 256

TQ = 512


def _bn_relu_body(x_ref, w_ref, b_ref, g_ref, be_ref, o_ref):
    x = jnp.dot(x_ref[...], w_ref[...], preferred_element_type=jnp.float32)
    x = x + b_ref[...]
    m = jnp.mean(x, axis=0, keepdims=True)
    v = jnp.mean((x - m) ** 2, axis=0, keepdims=True)
    o_ref[...] = jax.nn.relu((x - m) / jnp.sqrt(v + 1e-5) * g_ref[...] + be_ref[...])


def _branch(feats, w, b, g, be):
    rows, _ = feats.shape
    return pl.pallas_call(
        _bn_relu_body,
        out_shape=jax.ShapeDtypeStruct((rows, OUT), jnp.float32),
    )(feats, w, b.reshape(1, OUT), g.reshape(1, OUT), be.reshape(1, OUT))


def _search_interp_body(pos2_ref, b2_ref, pos1t_ref, b1_ref, f1_ref, f2_ref,
                        o_ref):
    rowmask = jax.lax.broadcasted_iota(jnp.int32, (8, N), 0) < 3
    pxt = jnp.where(rowmask, pos1t_ref[...] + b1_ref[...] * 1000.0, 0.0)
    colmask = jax.lax.broadcasted_iota(jnp.int32, (TQ, 8), 1) < 3
    pyp = jnp.where(colmask, pos2_ref[...] + b2_ref[...] * 1000.0, 0.0)

    s2 = (pxt[0:1] * pxt[0:1] + pxt[1:2] * pxt[1:2]) + pxt[2:3] * pxt[2:3]
    s1 = ((pyp[:, 0:1] * pyp[:, 0:1] + pyp[:, 1:2] * pyp[:, 1:2])
          + pyp[:, 2:3] * pyp[:, 2:3])
    dot = jax.lax.dot_general(pyp, pxt, (((1,), (0,)), ((), ())),
                              preferred_element_type=jnp.float32)
    d2 = (s1 + s2) - 2.0 * dot

    iota = jax.lax.broadcasted_iota(jnp.int32, (TQ, N), 1)
    work = d2
    wsum = None
    sel = jnp.zeros((TQ, N), jnp.float32)
    for k in range(3):
        m = jnp.min(work, axis=1, keepdims=True)
        idx = jnp.min(jnp.where(work == m, iota, N), axis=1, keepdims=True)
        wk = 1.0 / jnp.maximum(jnp.maximum(m, 0.0), 1e-16)
        wsum = wk if wsum is None else wsum + wk
        sel = sel + jnp.where(iota == idx, wk, 0.0)
        if k < 2:
            work = jnp.where(iota == idx, jnp.float32(jnp.inf), work)

    num = jax.lax.dot_general(sel, f1_ref[...], (((1,), (0,)), ((), ())),
                              precision=jax.lax.Precision.HIGHEST,
                              preferred_element_type=jnp.float32)
    o_ref[...] = num / wsum + f2_ref[...]


def kernel(features_1, positions_1, batch_1, features_2, positions_2, batch_2,
           W1, b1, g1, be1, W2, b2, g2, be2):
    f1 = _branch(features_1, W1, b1, g1, be1)
    f2 = _branch(features_2, W2, b2, g2, be2)

    pos1t = jnp.zeros((8, N), jnp.float32).at[:3].set(positions_1.T)
    b1f = batch_1.astype(jnp.float32).reshape(1, N)
    pos2p = jnp.zeros((L, 8), jnp.float32).at[:, :3].set(positions_2)
    b2f = batch_2.astype(jnp.float32).reshape(L, 1)

    grid = L // TQ
    out = pl.pallas_call(
        _search_interp_body,
        grid=(grid,),
        in_specs=[
            pl.BlockSpec((TQ, 8), lambda i: (i, 0)),
            pl.BlockSpec((TQ, 1), lambda i: (i, 0)),
            pl.BlockSpec((8, N), lambda i: (0, 0)),
            pl.BlockSpec((1, N), lambda i: (0, 0)),
            pl.BlockSpec((N, OUT), lambda i: (0, 0)),
            pl.BlockSpec((TQ, OUT), lambda i: (i, 0)),
        ],
        out_specs=pl.BlockSpec((TQ, OUT), lambda i: (i, 0)),
        out_shape=jax.ShapeDtypeStruct((L, OUT), jnp.float32),
    )(pos2p, b2f, pos1t, b1f, f1, f2)

    return (out, positions_2, batch_2)
